# baseline (speedup 1.0000x reference)
import functools

import jax
import jax.numpy as jnp
from jax.experimental import pallas as pl
from jax.experimental.pallas import tpu as pltpu

_EPS = 1e-6
_CH = 16


def _grn_slab_compute(in_ref, out_ref, gamma_ref, beta_ref, islot, oslot,
                      *, inv_d):
    _, bb, t, d = in_ref.shape
    n_chunks = t // _CH

    acc = jnp.zeros((bb, _CH, d), jnp.float32)
    for k in range(n_chunks):
        c = in_ref[islot, :, k * _CH:(k + 1) * _CH, :]
        acc += c * c
    ssq = jnp.sum(acc, axis=1, keepdims=True)

    gx = jnp.sqrt(ssq)
    mean = jnp.sum(gx, axis=-1, keepdims=True) * inv_d
    scale = gamma_ref[...] * (gx / (mean + _EPS)) + 1.0
    beta = beta_ref[...]

    for k in range(n_chunks):
        sl = pl.ds(k * _CH, _CH)
        out_ref[oslot, :, sl, :] = in_ref[islot, :, sl, :] * scale + beta


def _grn_manual_kernel(x_hbm, gamma_ref, beta_ref, o_hbm,
                       in_buf, out_buf, in_sem, out_sem,
                       *, inv_d, slabs_per_core, bb):
    core = pl.program_id(0)
    base = core * slabs_per_core

    def in_copy(r, j):
        return pltpu.make_async_copy(
            x_hbm.at[pl.ds((base + r) * bb + j, 1)],
            in_buf.at[r % 3, pl.ds(j, 1)],
            in_sem.at[r % 3, j],
        )

    def out_copy(r, j):
        return pltpu.make_async_copy(
            out_buf.at[r % 2, pl.ds(j, 1)],
            o_hbm.at[pl.ds((base + r) * bb + j, 1)],
            out_sem.at[r % 2, j],
        )

    for r in (0, 1):
        for j in range(bb):
            in_copy(r, j).start()

    for r in range(slabs_per_core):
        for j in range(bb):
            in_copy(r, j).wait()
        if r + 2 < slabs_per_core:
            for j in range(bb):
                in_copy(r + 2, j).start()
        if r >= 2:
            for j in range(bb):
                out_copy(r - 2, j).wait()
        _grn_slab_compute(in_buf, out_buf, gamma_ref, beta_ref,
                          r % 3, r % 2, inv_d=inv_d)
        for j in range(bb):
            out_copy(r, j).start()

    for r in (slabs_per_core - 2, slabs_per_core - 1):
        for j in range(bb):
            out_copy(r, j).wait()


def kernel(x, gamma, beta):
    B, T, D = x.shape
    g = gamma.reshape(1, 1, D).astype(jnp.float32)
    b = beta.reshape(1, 1, D).astype(jnp.float32)

    Bb = 2
    n_cores = 2
    slabs_per_core = B // (Bb * n_cores)

    return pl.pallas_call(
        functools.partial(_grn_manual_kernel, inv_d=1.0 / D,
                          slabs_per_core=slabs_per_core, bb=Bb),
        out_shape=jax.ShapeDtypeStruct((B, T, D), x.dtype),
        grid=(n_cores,),
        in_specs=[
            pl.BlockSpec(memory_space=pl.ANY),
            pl.BlockSpec((1, 1, D), lambda c: (0, 0, 0)),
            pl.BlockSpec((1, 1, D), lambda c: (0, 0, 0)),
        ],
        out_specs=pl.BlockSpec(memory_space=pl.ANY),
        scratch_shapes=[
            pltpu.VMEM((3, Bb, T, D), jnp.float32),
            pltpu.VMEM((2, Bb, T, D), jnp.float32),
            pltpu.SemaphoreType.DMA((3, Bb)),
            pltpu.SemaphoreType.DMA((2, Bb)),
        ],
        compiler_params=pltpu.CompilerParams(
            dimension_semantics=("parallel",),
            vmem_limit_bytes=58 << 20,
        ),
    )(x.astype(jnp.float32), g, b)


# speedup vs baseline: 1.0051x; 1.0051x over previous
import functools

import jax
import jax.numpy as jnp
from jax.experimental import pallas as pl
from jax.experimental.pallas import tpu as pltpu

_EPS = 1e-6
_CH = 16


def _grn_slab_compute(in_ref, out_ref, gamma_ref, beta_ref, islot, oslot,
                      *, inv_d):
    _, bb, t, d = in_ref.shape
    n_chunks = t // _CH

    acc = jnp.zeros((bb, _CH, d), jnp.float32)
    for k in range(n_chunks):
        c = in_ref[islot, :, k * _CH:(k + 1) * _CH, :]
        acc += c * c
    ssq = jnp.sum(acc, axis=1, keepdims=True)

    gx = jnp.sqrt(ssq)
    mean = jnp.sum(gx, axis=-1, keepdims=True) * inv_d
    scale = gamma_ref[...] * (gx / (mean + _EPS)) + 1.0
    beta = beta_ref[...]

    for k in range(n_chunks):
        sl = pl.ds(k * _CH, _CH)
        out_ref[oslot, :, sl, :] = in_ref[islot, :, sl, :] * scale + beta


def _grn_manual_kernel(x_hbm, gamma_ref, beta_ref, o_hbm,
                       in_buf, out_buf, in_sem, out_sem,
                       *, inv_d, slabs_per_core, bb):
    core = pl.program_id(0)
    base = core * slabs_per_core

    def in_copy(r):
        return pltpu.make_async_copy(
            x_hbm.at[pl.ds((base + r) * bb, bb)],
            in_buf.at[r % 4],
            in_sem.at[r % 4],
        )

    def out_copy(r):
        return pltpu.make_async_copy(
            out_buf.at[r % 2],
            o_hbm.at[pl.ds((base + r) * bb, bb)],
            out_sem.at[r % 2],
        )

    for r in range(min(3, slabs_per_core)):
        in_copy(r).start()

    for r in range(slabs_per_core):
        in_copy(r).wait()
        if r + 3 < slabs_per_core:
            in_copy(r + 3).start()
        if r >= 2:
            out_copy(r - 2).wait()
        _grn_slab_compute(in_buf, out_buf, gamma_ref, beta_ref,
                          r % 4, r % 2, inv_d=inv_d)
        out_copy(r).start()

    out_copy(slabs_per_core - 2).wait()
    out_copy(slabs_per_core - 1).wait()


def kernel(x, gamma, beta):
    B, T, D = x.shape
    g = gamma.reshape(1, 1, D).astype(jnp.float32)
    b = beta.reshape(1, 1, D).astype(jnp.float32)

    Bb = 2
    n_cores = 2
    slabs_per_core = B // (Bb * n_cores)

    return pl.pallas_call(
        functools.partial(_grn_manual_kernel, inv_d=1.0 / D,
                          slabs_per_core=slabs_per_core, bb=Bb),
        out_shape=jax.ShapeDtypeStruct((B, T, D), x.dtype),
        grid=(n_cores,),
        in_specs=[
            pl.BlockSpec(memory_space=pl.ANY),
            pl.BlockSpec((1, 1, D), lambda c: (0, 0, 0)),
            pl.BlockSpec((1, 1, D), lambda c: (0, 0, 0)),
        ],
        out_specs=pl.BlockSpec(memory_space=pl.ANY),
        scratch_shapes=[
            pltpu.VMEM((4, Bb, T, D), jnp.float32),
            pltpu.VMEM((2, Bb, T, D), jnp.float32),
            pltpu.SemaphoreType.DMA((4,)),
            pltpu.SemaphoreType.DMA((2,)),
        ],
        compiler_params=pltpu.CompilerParams(
            dimension_semantics=("parallel",),
            vmem_limit_bytes=58 << 20,
        ),
    )(x.astype(jnp.float32), g, b)
